# baseline (speedup 1.0000x reference)
import jax
import jax.numpy as jnp
from jax.experimental import pallas as pl
from jax.experimental.pallas import tpu as pltpu

_B = 1024
_NF = 100000
_D = 64
_H1 = 64
_H2 = 32
_KT = 2048
_NB = (_NF + _KT - 1) // _KT
_EPS = 1e-5

_TDOT = (((0,), (1,)), ((), ()))


def _bn_t(v, g, b):
    mu = jnp.mean(v, axis=1, keepdims=True)
    var = jnp.mean(jnp.square(v - mu), axis=1, keepdims=True)
    return (v - mu) / jnp.sqrt(var + _EPS) * g + b


def _nfm_kernel(x_ref, e_ref, lw_ref, lb_ref, g0_ref, b0_ref,
                w1_ref, b1_ref, g1_ref, bb1_ref,
                w2_ref, b2_ref, g2_ref, bb2_ref, hw_ref,
                out_ref, acc_a, acc_q):
    k = pl.program_id(0)

    @pl.when(k == 0)
    def _init():
        acc_a[...] = jnp.zeros_like(acc_a)
        acc_q[...] = jnp.zeros_like(acc_q)

    def _accumulate(x, e, lw):
        aug = jnp.concatenate([e, lw], axis=1)
        acc_a[...] += jax.lax.dot_general(
            aug, x, _TDOT, preferred_element_type=jnp.float32)
        acc_q[...] += jax.lax.dot_general(
            e * e, x * x, _TDOT, preferred_element_type=jnp.float32)

    @pl.when(k < _NB - 1)
    def _full_tile():
        _accumulate(x_ref[...].astype(jnp.bfloat16),
                    e_ref[...].astype(jnp.bfloat16),
                    lw_ref[...].astype(jnp.bfloat16))

    @pl.when(k == _NB - 1)
    def _partial_tile():
        nvalid = _NF - (_NB - 1) * _KT
        lane = jax.lax.broadcasted_iota(jnp.int32, (1, _KT), 1)
        sub = jax.lax.broadcasted_iota(jnp.int32, (_KT, 1), 0)
        x = jnp.where(lane < nvalid, x_ref[...], 0.0).astype(jnp.bfloat16)
        e = jnp.where(sub < nvalid, e_ref[...], 0.0).astype(jnp.bfloat16)
        lw = jnp.where(sub < nvalid, lw_ref[...], 0.0).astype(jnp.bfloat16)
        _accumulate(x, e, lw)

        se = acc_a[:_D, :]
        lin = acc_a[_D:_D + 1, :]
        bi = 0.5 * (se * se - acc_q[...])
        z = _bn_t(bi, g0_ref[...], b0_ref[...])
        z = jnp.dot(w1_ref[...], z,
                    preferred_element_type=jnp.float32) + b1_ref[...]
        z = jax.nn.relu(_bn_t(z, g1_ref[...], bb1_ref[...]))
        z = jnp.dot(w2_ref[...], z,
                    preferred_element_type=jnp.float32) + b2_ref[...]
        z = jax.nn.relu(_bn_t(z, g2_ref[...], bb2_ref[...]))
        y = jnp.sum(z * hw_ref[...], axis=0, keepdims=True)
        out_ref[...] = y + lin + lb_ref[...]


def kernel(feature_values, feature_embed, lin_w, lin_b, bn0_g, bn0_b,
           W1, b1, bn1_g, bn1_b, W2, b2, bn2_g, bn2_b, h_w):
    out = pl.pallas_call(
        _nfm_kernel,
        grid=(_NB,),
        in_specs=[
            pl.BlockSpec((_B, _KT), lambda k: (0, k)),
            pl.BlockSpec((_KT, _D), lambda k: (k, 0)),
            pl.BlockSpec((_KT, 1), lambda k: (k, 0)),
            pl.BlockSpec((1, 1), lambda k: (0, 0)),
            pl.BlockSpec((_D, 1), lambda k: (0, 0)),
            pl.BlockSpec((_D, 1), lambda k: (0, 0)),
            pl.BlockSpec((_H1, _D), lambda k: (0, 0)),
            pl.BlockSpec((_H1, 1), lambda k: (0, 0)),
            pl.BlockSpec((_H1, 1), lambda k: (0, 0)),
            pl.BlockSpec((_H1, 1), lambda k: (0, 0)),
            pl.BlockSpec((_H2, _H1), lambda k: (0, 0)),
            pl.BlockSpec((_H2, 1), lambda k: (0, 0)),
            pl.BlockSpec((_H2, 1), lambda k: (0, 0)),
            pl.BlockSpec((_H2, 1), lambda k: (0, 0)),
            pl.BlockSpec((_H2, 1), lambda k: (0, 0)),
        ],
        out_specs=pl.BlockSpec((1, _B), lambda k: (0, 0)),
        out_shape=jax.ShapeDtypeStruct((1, _B), jnp.float32),
        scratch_shapes=[
            pltpu.VMEM((_D + 1, _B), jnp.float32),
            pltpu.VMEM((_D, _B), jnp.float32),
        ],
        compiler_params=pltpu.CompilerParams(
            dimension_semantics=("arbitrary",),
        ),
    )(feature_values, feature_embed, lin_w.reshape(_NF, 1),
      lin_b.reshape(1, 1), bn0_g.reshape(_D, 1), bn0_b.reshape(_D, 1),
      W1, b1.reshape(_H1, 1), bn1_g.reshape(_H1, 1), bn1_b.reshape(_H1, 1),
      W2, b2.reshape(_H2, 1), bn2_g.reshape(_H2, 1), bn2_b.reshape(_H2, 1),
      h_w.reshape(_H2, 1))
    return out.reshape(_B)
